# baseline (speedup 1.0000x reference)
import functools

import jax
import jax.numpy as jnp
from jax import lax
from jax.experimental import pallas as pl
from jax.experimental.pallas import tpu as pltpu
from jax.experimental.pallas import tpu_sc as plsc

N_NODES = 10000
N_PAD = 10240
N_EDGES = 320000
C = 128
H = 64
NS = 16
B = 128
SL = 16
NBUF = 4
R = 160
E_PAD = NS * R * B
ROWS_PER_TILE = N_PAD // NS


def _matmul_body(x_ref, w_ref, o_ref):
    o_ref[...] = jnp.dot(x_ref[...], w_ref[...],
                         preferred_element_type=jnp.float32)


def _tc_matmul(features, weight_matrix):
    return pl.pallas_call(
        _matmul_body,
        grid=(10,),
        in_specs=[
            pl.BlockSpec((N_NODES // 10, C), lambda i: (i, 0)),
            pl.BlockSpec((C, C), lambda i: (0, 0)),
        ],
        out_specs=pl.BlockSpec((N_NODES // 10, C), lambda i: (i, 0)),
        out_shape=jax.ShapeDtypeStruct((N_NODES, C), jnp.float32),
    )(features, weight_matrix)


def _sc_body(row_hbm, col_hbm, val_hbm, base2_hbm, bias_hbm, out_hbm,
             acc1, acc2, rslab, cslab, c2slab, vslab, g0, g1, g2, g3,
             bbuf, widx, sem0, gsem0, gsem1, gsem2, gsem3,
             ssem0, ssem1, ssem2, ssem3):
    c = lax.axis_index("c")
    s = lax.axis_index("s")
    iota16 = lax.iota(jnp.int32, 16)

    pltpu.sync_copy(bias_hbm.at[pl.ds(c * H, H)], bbuf)

    nbase = s * ROWS_PER_TILE
    for k in range(ROWS_PER_TILE // B):
        for m in range(B // 16):
            widx[k, pl.ds(16 * m, 16)] = (
                (nbase + B * k + 16 * m + iota16) * 2 + c)

    zero16 = jnp.zeros((16,), jnp.float32)

    def _fill_zero(j, _):
        for g in range(H // 16):
            g0[j, pl.ds(16 * g, 16)] = zero16
        return 0
    lax.fori_loop(0, B, _fill_zero, 0)

    def _fill_bias(j, _):
        for g in range(H // 16):
            g1[j, pl.ds(16 * g, 16)] = bbuf[pl.ds(16 * g, 16)]
        return 0
    lax.fori_loop(0, B, _fill_bias, 0)

    for k in range(ROWS_PER_TILE // B):
        pltpu.sync_copy(g0, acc1.at[pl.ds(nbase + B * k, B)])
        pltpu.sync_copy(g1, acc2.at[pl.ds(nbase + B * k, B)])
    plsc.subcore_barrier()

    def _scale(j, g):
        def body(eb, _):
            vv = vslab[j, pl.ds(16 * eb, 16)]
            for ee in range(16):
                e = 16 * eb + ee
                v = vv[ee]
                for gg in range(H // 16):
                    sl = pl.ds(16 * gg, 16)
                    g[e, sl] = g[e, sl] * v
            return 0
        lax.fori_loop(0, B // 16, body, 0)

    def _pass(src_ref, idx_slab, use_c2, dst_acc):
        gbufs = (g0, g1, g2, g3)
        gsems = (gsem0, gsem1, gsem2, gsem3)
        ssems = (ssem0, ssem1, ssem2, ssem3)

        def slab(m, _):
            rowbase = s * R + SL * m
            pltpu.sync_copy(row_hbm.at[pl.ds(rowbase, SL)], rslab)
            pltpu.sync_copy(col_hbm.at[pl.ds(rowbase, SL)], cslab)
            pltpu.sync_copy(val_hbm.at[pl.ds(rowbase, SL)], vslab)
            if use_c2:
                def mkidx(j, _):
                    for g in range(B // 16):
                        cv = cslab[j, pl.ds(16 * g, 16)]
                        c2slab[j, pl.ds(16 * g, 16)] = cv * 2 + c
                    return 0
                lax.fori_loop(0, SL, mkidx, 0)

            for p in range(NBUF - 1):
                pltpu.async_copy(src_ref.at[idx_slab.at[p]], gbufs[p],
                                 gsems[p])

            def quad(q, _):
                for b4 in range(NBUF):
                    j = NBUF * q + b4
                    bi = b4
                    nb = (b4 + NBUF - 1) % NBUF

                    @pl.when(j >= 1)
                    def _():
                        pltpu.make_async_copy(
                            gbufs[nb], dst_acc.at[rslab.at[j - 1]],
                            ssems[nb]).wait()

                    @pl.when(j + NBUF - 1 < SL)
                    def _():
                        pltpu.async_copy(
                            src_ref.at[idx_slab.at[j + NBUF - 1]],
                            gbufs[nb], gsems[nb])

                    pltpu.make_async_copy(
                        src_ref.at[idx_slab.at[j]], gbufs[bi],
                        gsems[bi]).wait()
                    _scale(j, gbufs[bi])
                    pltpu.async_copy(
                        gbufs[bi], dst_acc.at[rslab.at[j]], ssems[bi],
                        add=True)
                return 0
            lax.fori_loop(0, SL // NBUF, quad, 0)
            pltpu.make_async_copy(
                gbufs[(SL - 1) % NBUF], dst_acc.at[rslab.at[SL - 1]],
                ssems[(SL - 1) % NBUF]).wait()
            return 0
        lax.fori_loop(0, R // SL, slab, 0)

    _pass(base2_hbm, c2slab, True, acc1)
    plsc.subcore_barrier()
    _pass(acc1, cslab, False, acc2)
    plsc.subcore_barrier()

    for k in range(ROWS_PER_TILE // B):
        pltpu.sync_copy(acc2.at[pl.ds(nbase + B * k, B)], g0)
        pltpu.async_copy(g0, out_hbm.at[widx.at[k]], sem0).wait()


_sc_kernel = functools.partial(
    pl.kernel,
    out_type=jax.ShapeDtypeStruct((2 * N_PAD, H), jnp.float32),
    mesh=plsc.VectorSubcoreMesh(core_axis_name="c", subcore_axis_name="s"),
    compiler_params=pltpu.CompilerParams(use_tc_tiling_on_sc=False),
    scratch_types=[
        pltpu.VMEM_SHARED((N_PAD, H), jnp.float32),
        pltpu.VMEM_SHARED((N_PAD, H), jnp.float32),
        pltpu.VMEM((SL, B), jnp.int32),
        pltpu.VMEM((SL, B), jnp.int32),
        pltpu.VMEM((SL, B), jnp.int32),
        pltpu.VMEM((SL, B), jnp.float32),
        pltpu.VMEM((B, H), jnp.float32),
        pltpu.VMEM((B, H), jnp.float32),
        pltpu.VMEM((B, H), jnp.float32),
        pltpu.VMEM((B, H), jnp.float32),
        pltpu.VMEM((H,), jnp.float32),
        pltpu.VMEM((ROWS_PER_TILE // B, B), jnp.int32),
        pltpu.SemaphoreType.DMA,
        pltpu.SemaphoreType.DMA,
        pltpu.SemaphoreType.DMA,
        pltpu.SemaphoreType.DMA,
        pltpu.SemaphoreType.DMA,
        pltpu.SemaphoreType.DMA,
        pltpu.SemaphoreType.DMA,
        pltpu.SemaphoreType.DMA,
        pltpu.SemaphoreType.DMA,
    ],
)(_sc_body)


def kernel(normalized_adjacency_matrix_indices, normalized_adjacency_matrix_values,
           features, weight_matrix, bias):
    base = _tc_matmul(features, weight_matrix)

    row = normalized_adjacency_matrix_indices[0]
    col = normalized_adjacency_matrix_indices[1]
    pad = E_PAD - N_EDGES
    rowp = jnp.pad(row, (0, pad)).reshape(NS * R, B)
    colp = jnp.pad(col, (0, pad)).reshape(NS * R, B)
    valp = jnp.pad(normalized_adjacency_matrix_values, (0, pad)).reshape(NS * R, B)
    base2 = base.reshape(2 * N_NODES, H)
    bias1 = bias.reshape(C)

    out2 = _sc_kernel(rowp, colp, valp, base2, bias1)
    return out2.reshape(N_PAD, C)[:N_NODES]


# speedup vs baseline: 1.3988x; 1.3988x over previous
import functools

import jax
import jax.numpy as jnp
from jax import lax
from jax.experimental import pallas as pl
from jax.experimental.pallas import tpu as pltpu
from jax.experimental.pallas import tpu_sc as plsc

N_NODES = 10000
N_PAD = 10240
N_EDGES = 320000
C = 128
H = 64
NS = 16
B = 128
SL = 32
NBUF = 4
R = 160
E_PAD = NS * R * B
ROWS_PER_TILE = N_PAD // NS
KW = ROWS_PER_TILE // B


def _matmul_body(x_ref, w_ref, o_ref):
    o_ref[...] = jnp.dot(x_ref[...], w_ref[...],
                         preferred_element_type=jnp.float32)


def _tc_matmul(features, weight_matrix):
    return pl.pallas_call(
        _matmul_body,
        grid=(10,),
        in_specs=[
            pl.BlockSpec((N_PAD // 10, C), lambda i: (i, 0)),
            pl.BlockSpec((C, C), lambda i: (0, 0)),
        ],
        out_specs=pl.BlockSpec((N_PAD // 10, C), lambda i: (i, 0)),
        out_shape=jax.ShapeDtypeStruct((N_PAD, C), jnp.float32),
    )(features, weight_matrix)


def _sc_body(row_hbm, col_hbm, val_hbm, base2_hbm, bias_hbm, out_hbm,
             srcb, acc, rslab, cslab, vslab, g0, g1, g2, g3,
             bbuf, widx, sem0, gsem0, gsem1, gsem2, gsem3,
             ssem0, ssem1, ssem2, ssem3):
    c = lax.axis_index("c")
    s = lax.axis_index("s")
    iota16 = lax.iota(jnp.int32, 16)

    pltpu.sync_copy(bias_hbm.at[pl.ds(c * H, H)], bbuf)

    nbase = s * ROWS_PER_TILE
    for k in range(KW):
        for m in range(B // 16):
            widx[k, pl.ds(16 * m, 16)] = (
                (nbase + B * k + 16 * m + iota16) * 2 + c)

    zero16 = jnp.zeros((16,), jnp.float32)

    def _fill_zero(j, _):
        for g in range(H // 16):
            g1[j, pl.ds(16 * g, 16)] = zero16
        return 0
    lax.fori_loop(0, B, _fill_zero, 0)

    for k in range(KW):
        pltpu.async_copy(base2_hbm.at[widx.at[k]], g0, sem0).wait()
        pltpu.sync_copy(g0, srcb.at[pl.ds(nbase + B * k, B)])
        pltpu.sync_copy(g1, acc.at[pl.ds(nbase + B * k, B)])
    plsc.subcore_barrier()

    def _scale(j, g):
        def body(eb, _):
            vv = vslab[j, pl.ds(16 * eb, 16)]
            for ee in range(16):
                e = 16 * eb + ee
                v = vv[ee]
                for gg in range(H // 16):
                    sl = pl.ds(16 * gg, 16)
                    g[e, sl] = g[e, sl] * v
            return 0
        lax.fori_loop(0, B // 16, body, 0)

    def _pass(src_ref, dst_acc):
        gbufs = (g0, g1, g2, g3)
        gsems = (gsem0, gsem1, gsem2, gsem3)
        ssems = (ssem0, ssem1, ssem2, ssem3)

        def slab(m, _):
            rowbase = s * R + SL * m
            pltpu.sync_copy(row_hbm.at[pl.ds(rowbase, SL)], rslab)
            pltpu.sync_copy(col_hbm.at[pl.ds(rowbase, SL)], cslab)
            pltpu.sync_copy(val_hbm.at[pl.ds(rowbase, SL)], vslab)

            for p in range(NBUF - 1):
                pltpu.async_copy(src_ref.at[cslab.at[p]], gbufs[p],
                                 gsems[p])

            def quad(q, _):
                for b4 in range(NBUF):
                    j = NBUF * q + b4
                    bi = b4
                    nb = (b4 + NBUF - 1) % NBUF

                    @pl.when(j >= 1)
                    def _():
                        pltpu.make_async_copy(
                            gbufs[nb], dst_acc.at[rslab.at[j - 1]],
                            ssems[nb]).wait()

                    @pl.when(j + NBUF - 1 < SL)
                    def _():
                        pltpu.async_copy(
                            src_ref.at[cslab.at[j + NBUF - 1]],
                            gbufs[nb], gsems[nb])

                    pltpu.make_async_copy(
                        src_ref.at[cslab.at[j]], gbufs[bi],
                        gsems[bi]).wait()
                    _scale(j, gbufs[bi])
                    pltpu.async_copy(
                        gbufs[bi], dst_acc.at[rslab.at[j]], ssems[bi],
                        add=True)
                return 0
            lax.fori_loop(0, SL // NBUF, quad, 0)
            pltpu.make_async_copy(
                gbufs[(SL - 1) % NBUF], dst_acc.at[rslab.at[SL - 1]],
                ssems[(SL - 1) % NBUF]).wait()
            return 0
        lax.fori_loop(0, R // SL, slab, 0)

    _pass(srcb, acc)
    plsc.subcore_barrier()

    def _fill_bias(j, _):
        for g in range(H // 16):
            g1[j, pl.ds(16 * g, 16)] = bbuf[pl.ds(16 * g, 16)]
        return 0
    lax.fori_loop(0, B, _fill_bias, 0)
    for k in range(KW):
        pltpu.sync_copy(g1, srcb.at[pl.ds(nbase + B * k, B)])
    plsc.subcore_barrier()

    _pass(acc, srcb)
    plsc.subcore_barrier()

    for k in range(KW):
        pltpu.sync_copy(srcb.at[pl.ds(nbase + B * k, B)], g0)
        pltpu.async_copy(g0, out_hbm.at[widx.at[k]], sem0).wait()


_sc_kernel = functools.partial(
    pl.kernel,
    out_type=jax.ShapeDtypeStruct((2 * N_PAD, H), jnp.float32),
    mesh=plsc.VectorSubcoreMesh(core_axis_name="c", subcore_axis_name="s"),
    compiler_params=pltpu.CompilerParams(use_tc_tiling_on_sc=False),
    scratch_types=[
        pltpu.VMEM_SHARED((N_PAD, H), jnp.float32),
        pltpu.VMEM_SHARED((N_PAD, H), jnp.float32),
        pltpu.VMEM((SL, B), jnp.int32),
        pltpu.VMEM((SL, B), jnp.int32),
        pltpu.VMEM((SL, B), jnp.float32),
        pltpu.VMEM((B, H), jnp.float32),
        pltpu.VMEM((B, H), jnp.float32),
        pltpu.VMEM((B, H), jnp.float32),
        pltpu.VMEM((B, H), jnp.float32),
        pltpu.VMEM((H,), jnp.float32),
        pltpu.VMEM((KW, B), jnp.int32),
        pltpu.SemaphoreType.DMA,
        pltpu.SemaphoreType.DMA,
        pltpu.SemaphoreType.DMA,
        pltpu.SemaphoreType.DMA,
        pltpu.SemaphoreType.DMA,
        pltpu.SemaphoreType.DMA,
        pltpu.SemaphoreType.DMA,
        pltpu.SemaphoreType.DMA,
        pltpu.SemaphoreType.DMA,
    ],
)(_sc_body)


def kernel(normalized_adjacency_matrix_indices, normalized_adjacency_matrix_values,
           features, weight_matrix, bias):
    feats = jnp.pad(features, ((0, N_PAD - N_NODES), (0, 0)))
    base = _tc_matmul(feats, weight_matrix)

    row = normalized_adjacency_matrix_indices[0]
    col = normalized_adjacency_matrix_indices[1]
    pad = E_PAD - N_EDGES
    rowp = jnp.pad(row, (0, pad)).reshape(NS * R, B)
    colp = jnp.pad(col, (0, pad)).reshape(NS * R, B)
    valp = jnp.pad(normalized_adjacency_matrix_values, (0, pad)).reshape(NS * R, B)
    base2 = base.reshape(2 * N_PAD, H)
    bias1 = bias.reshape(C)

    out2 = _sc_kernel(rowp, colp, valp, base2, bias1)
    return out2.reshape(N_PAD, C)[:N_NODES]


# speedup vs baseline: 1.6188x; 1.1573x over previous
import functools

import jax
import jax.numpy as jnp
from jax import lax
from jax.experimental import pallas as pl
from jax.experimental.pallas import tpu as pltpu
from jax.experimental.pallas import tpu_sc as plsc

N_NODES = 10000
N_PAD = 10240
N_EDGES = 320000
C = 128
H = 64
NS = 16
B = 128
SL = 32
NBUF = 4
R = 160
E_PAD = NS * R * B
ROWS_PER_TILE = N_PAD // NS
KW = ROWS_PER_TILE // B


def _matmul_body(x_ref, w_ref, o_ref):
    o_ref[...] = jnp.dot(x_ref[...], w_ref[...],
                         preferred_element_type=jnp.float32)


def _tc_matmul(features, weight_matrix):
    return pl.pallas_call(
        _matmul_body,
        grid=(10,),
        in_specs=[
            pl.BlockSpec((N_PAD // 10, C), lambda i: (i, 0)),
            pl.BlockSpec((C, C), lambda i: (0, 0)),
        ],
        out_specs=pl.BlockSpec((N_PAD // 10, C), lambda i: (i, 0)),
        out_shape=jax.ShapeDtypeStruct((N_PAD, C), jnp.float32),
    )(features, weight_matrix)


def _sc_body(row_hbm, col_hbm, val_hbm, base2_hbm, bias_hbm, out_hbm,
             srcb, acc, rslab, cslab, vslab, g0, g1, g2, g3,
             bbuf, widx, sem0, gsem0, gsem1, gsem2, gsem3,
             ssem0, ssem1, ssem2, ssem3):
    c = lax.axis_index("c")
    s = lax.axis_index("s")
    iota16 = lax.iota(jnp.int32, 16)

    pltpu.sync_copy(bias_hbm.at[pl.ds(c * H, H)], bbuf)

    nbase = s * ROWS_PER_TILE
    for k in range(KW):
        for m in range(B // 16):
            widx[k, pl.ds(16 * m, 16)] = (
                (nbase + B * k + 16 * m + iota16) * 2 + c)

    zero16 = jnp.zeros((16,), jnp.float32)

    def _fill_zero(j, _):
        for g in range(H // 16):
            g1[j, pl.ds(16 * g, 16)] = zero16
        return 0
    lax.fori_loop(0, B, _fill_zero, 0)

    for k in range(KW):
        pltpu.async_copy(base2_hbm.at[widx.at[k]], g0, sem0).wait()
        pltpu.sync_copy(g0, srcb.at[pl.ds(nbase + B * k, B)])
        pltpu.sync_copy(g1, acc.at[pl.ds(nbase + B * k, B)])
    plsc.subcore_barrier()

    def _scale(j, g):
        @plsc.parallel_loop(0, B // 16, unroll=2)
        def _(eb):
            vv = vslab[j, pl.ds(16 * eb, 16)]
            for ee in range(16):
                e = 16 * eb + ee
                v = vv[ee]
                for gg in range(H // 16):
                    sl = pl.ds(16 * gg, 16)
                    g[e, sl] = g[e, sl] * v

    def _pass(src_ref, dst_acc):
        gbufs = (g0, g1, g2, g3)
        gsems = (gsem0, gsem1, gsem2, gsem3)
        ssems = (ssem0, ssem1, ssem2, ssem3)

        def slab(m, _):
            rowbase = s * R + SL * m
            pltpu.sync_copy(row_hbm.at[pl.ds(rowbase, SL)], rslab)
            pltpu.sync_copy(col_hbm.at[pl.ds(rowbase, SL)], cslab)
            pltpu.sync_copy(val_hbm.at[pl.ds(rowbase, SL)], vslab)

            for p in range(NBUF - 1):
                pltpu.async_copy(src_ref.at[cslab.at[p]], gbufs[p],
                                 gsems[p])

            def quad(q, _):
                for b4 in range(NBUF):
                    j = NBUF * q + b4
                    bi = b4
                    nb = (b4 + NBUF - 1) % NBUF

                    @pl.when(j >= 1)
                    def _():
                        pltpu.make_async_copy(
                            gbufs[nb], dst_acc.at[rslab.at[j - 1]],
                            ssems[nb]).wait()

                    @pl.when(j + NBUF - 1 < SL)
                    def _():
                        pltpu.async_copy(
                            src_ref.at[cslab.at[j + NBUF - 1]],
                            gbufs[nb], gsems[nb])

                    pltpu.make_async_copy(
                        src_ref.at[cslab.at[j]], gbufs[bi],
                        gsems[bi]).wait()
                    _scale(j, gbufs[bi])
                    pltpu.async_copy(
                        gbufs[bi], dst_acc.at[rslab.at[j]], ssems[bi],
                        add=True)
                return 0
            lax.fori_loop(0, SL // NBUF, quad, 0)
            pltpu.make_async_copy(
                gbufs[(SL - 1) % NBUF], dst_acc.at[rslab.at[SL - 1]],
                ssems[(SL - 1) % NBUF]).wait()
            return 0
        lax.fori_loop(0, R // SL, slab, 0)

    _pass(srcb, acc)
    plsc.subcore_barrier()

    def _fill_bias(j, _):
        for g in range(H // 16):
            g1[j, pl.ds(16 * g, 16)] = bbuf[pl.ds(16 * g, 16)]
        return 0
    lax.fori_loop(0, B, _fill_bias, 0)
    for k in range(KW):
        pltpu.sync_copy(g1, srcb.at[pl.ds(nbase + B * k, B)])
    plsc.subcore_barrier()

    _pass(acc, srcb)
    plsc.subcore_barrier()

    for k in range(KW):
        pltpu.sync_copy(srcb.at[pl.ds(nbase + B * k, B)], g0)
        pltpu.async_copy(g0, out_hbm.at[widx.at[k]], sem0).wait()


_sc_kernel = functools.partial(
    pl.kernel,
    out_type=jax.ShapeDtypeStruct((2 * N_PAD, H), jnp.float32),
    mesh=plsc.VectorSubcoreMesh(core_axis_name="c", subcore_axis_name="s"),
    compiler_params=pltpu.CompilerParams(use_tc_tiling_on_sc=False),
    scratch_types=[
        pltpu.VMEM_SHARED((N_PAD, H), jnp.float32),
        pltpu.VMEM_SHARED((N_PAD, H), jnp.float32),
        pltpu.VMEM((SL, B), jnp.int32),
        pltpu.VMEM((SL, B), jnp.int32),
        pltpu.VMEM((SL, B), jnp.float32),
        pltpu.VMEM((B, H), jnp.float32),
        pltpu.VMEM((B, H), jnp.float32),
        pltpu.VMEM((B, H), jnp.float32),
        pltpu.VMEM((B, H), jnp.float32),
        pltpu.VMEM((H,), jnp.float32),
        pltpu.VMEM((KW, B), jnp.int32),
        pltpu.SemaphoreType.DMA,
        pltpu.SemaphoreType.DMA,
        pltpu.SemaphoreType.DMA,
        pltpu.SemaphoreType.DMA,
        pltpu.SemaphoreType.DMA,
        pltpu.SemaphoreType.DMA,
        pltpu.SemaphoreType.DMA,
        pltpu.SemaphoreType.DMA,
        pltpu.SemaphoreType.DMA,
    ],
)(_sc_body)


def kernel(normalized_adjacency_matrix_indices, normalized_adjacency_matrix_values,
           features, weight_matrix, bias):
    feats = jnp.pad(features, ((0, N_PAD - N_NODES), (0, 0)))
    base = _tc_matmul(feats, weight_matrix)

    row = normalized_adjacency_matrix_indices[0]
    col = normalized_adjacency_matrix_indices[1]
    pad = E_PAD - N_EDGES
    rowp = jnp.pad(row, (0, pad)).reshape(NS * R, B)
    colp = jnp.pad(col, (0, pad)).reshape(NS * R, B)
    valp = jnp.pad(normalized_adjacency_matrix_values, (0, pad)).reshape(NS * R, B)
    base2 = base.reshape(2 * N_PAD, H)
    bias1 = bias.reshape(C)

    out2 = _sc_kernel(rowp, colp, valp, base2, bias1)
    return out2.reshape(N_PAD, C)[:N_NODES]
